# baseline (speedup 1.0000x reference)
import functools

import jax
import jax.numpy as jnp
from jax import lax
from jax.experimental import pallas as pl
from jax.experimental.pallas import tpu as pltpu
from jax.experimental.pallas import tpu_sc as plsc

N = 10000
E = 160000
DIN = 128
C = 256
HALF = 128
OUT = 64

NC = 2
NS = 16
BN = 1000
NB = N // BN
EPT = E // NS
CHUNK = 80
NCH = EPT // CHUNK
NCHP = 128
NPH = 2
PCH = NCHP // NPH
NA = N + 8
ZPT = 624

_f32 = jnp.float32


def _enc_body(seed_ref, x_ref, we_ref, be_ref, ide_ref, out_ref):
    h = jnp.dot(x_ref[...], we_ref[...], preferred_element_type=_f32)
    h = h + be_ref[...]
    i = pl.program_id(0)
    rows = i * BN + lax.broadcasted_iota(jnp.int32, (BN, 1), 0)
    is_seed = (rows < seed_ref[0]).astype(_f32)
    h = h + is_seed * ide_ref[...]
    out_ref[0] = h[:, :HALF]
    out_ref[1] = h[:, HALF:]


def _encode(x, seed, W_enc, b_enc, id_emb):
    return pl.pallas_call(
        _enc_body,
        grid_spec=pltpu.PrefetchScalarGridSpec(
            num_scalar_prefetch=1,
            grid=(NB,),
            in_specs=[
                pl.BlockSpec((BN, DIN), lambda i, *_: (i, 0)),
                pl.BlockSpec((DIN, C), lambda i, *_: (0, 0)),
                pl.BlockSpec((1, C), lambda i, *_: (0, 0)),
                pl.BlockSpec((1, C), lambda i, *_: (0, 0)),
            ],
            out_specs=pl.BlockSpec((2, BN, HALF), lambda i, *_: (0, i, 0)),
        ),
        out_shape=jax.ShapeDtypeStruct((2, N, HALF), _f32),
    )(seed, x, W_enc, b_enc.reshape(1, C), id_emb)


def _sage_body(h_ref, a_ref, ws_ref, wn_ref, b_ref, out_ref, *, relu):
    z = jnp.dot(h_ref[0], ws_ref[:HALF, :], preferred_element_type=_f32)
    z += jnp.dot(h_ref[1], ws_ref[HALF:, :], preferred_element_type=_f32)
    z += jnp.dot(a_ref[0], wn_ref[:HALF, :], preferred_element_type=_f32)
    z += jnp.dot(a_ref[1], wn_ref[HALF:, :], preferred_element_type=_f32)
    z += b_ref[...]
    if relu:
        z = jnp.maximum(z, 0.0)
    out_ref[0] = z[:, :HALF]
    out_ref[1] = z[:, HALF:]


def _sage(h_split, a_split, W_self, W_neigh, b, relu):
    return pl.pallas_call(
        functools.partial(_sage_body, relu=relu),
        grid=(NB,),
        in_specs=[
            pl.BlockSpec((2, BN, HALF), lambda i: (0, i, 0)),
            pl.BlockSpec((2, BN, HALF), lambda i: (0, i, 0)),
            pl.BlockSpec((C, C), lambda i: (0, 0)),
            pl.BlockSpec((C, C), lambda i: (0, 0)),
            pl.BlockSpec((1, C), lambda i: (0, 0)),
        ],
        out_specs=pl.BlockSpec((2, BN, HALF), lambda i: (0, i, 0)),
        out_shape=jax.ShapeDtypeStruct((2, N, HALF), _f32),
    )(h_split, a_split, W_self, W_neigh, b.reshape(1, C))


def _final_body(h_ref, a_ref, ws_ref, wn_ref, b_ref, lns_ref, lnb_ref,
                wh_ref, bh_ref, out_ref):
    z = jnp.dot(h_ref[0], ws_ref[:HALF, :], preferred_element_type=_f32)
    z += jnp.dot(h_ref[1], ws_ref[HALF:, :], preferred_element_type=_f32)
    z += jnp.dot(a_ref[0], wn_ref[:HALF, :], preferred_element_type=_f32)
    z += jnp.dot(a_ref[1], wn_ref[HALF:, :], preferred_element_type=_f32)
    z += b_ref[...]
    mu = jnp.mean(z, axis=-1, keepdims=True)
    var = jnp.mean((z - mu) * (z - mu), axis=-1, keepdims=True)
    hn = (z - mu) * lax.rsqrt(var + 1e-5) * lns_ref[...] + lnb_ref[...]
    out_ref[...] = jnp.dot(hn, wh_ref[...], preferred_element_type=_f32)
    out_ref[...] += bh_ref[...]


def _final(h_split, a_split, W_self, W_neigh, b, ln_scale, ln_bias,
           W_head, b_head):
    return pl.pallas_call(
        _final_body,
        grid=(NB,),
        in_specs=[
            pl.BlockSpec((2, BN, HALF), lambda i: (0, i, 0)),
            pl.BlockSpec((2, BN, HALF), lambda i: (0, i, 0)),
            pl.BlockSpec((C, C), lambda i: (0, 0)),
            pl.BlockSpec((C, C), lambda i: (0, 0)),
            pl.BlockSpec((1, C), lambda i: (0, 0)),
            pl.BlockSpec((1, C), lambda i: (0, 0)),
            pl.BlockSpec((1, C), lambda i: (0, 0)),
            pl.BlockSpec((C, OUT), lambda i: (0, 0)),
            pl.BlockSpec((1, OUT), lambda i: (0, 0)),
        ],
        out_specs=pl.BlockSpec((BN, OUT), lambda i: (i, 0)),
        out_shape=jax.ShapeDtypeStruct((N, OUT), _f32),
    )(h_split, a_split, W_self, W_neigh, b.reshape(1, C),
      ln_scale.reshape(1, C), ln_bias.reshape(1, C), W_head,
      b_head.reshape(1, OUT))


def _segsum_body(h_hbm, srcp_hbm, dstp_hbm, zeros_hbm, out_hbm,
                 srcv, dstv, rows0, rows1, agg_sh, sem0, sem1, sem2, sem3):
    c = lax.axis_index("c")
    s = lax.axis_index("s")
    zrow = s * ZPT
    tail = NS * ZPT
    pltpu.sync_copy(zeros_hbm.at[pl.ds(zrow, ZPT)], agg_sh.at[pl.ds(zrow, ZPT)])

    @pl.when(s == NS - 1)
    def _():
        pltpu.sync_copy(zeros_hbm.at[pl.ds(tail, NA - tail)],
                        agg_sh.at[pl.ds(tail, NA - tail)])
    plsc.subcore_barrier()

    bufs = (rows0, rows1)
    gsems = (sem0, sem1)
    ssems = (sem2, sem3)
    for p in range(NPH):
        pltpu.sync_copy(srcp_hbm.at[(c * NS + s) * NPH + p], srcv)
        pltpu.sync_copy(dstp_hbm.at[s * NPH + p], dstv)
        pltpu.async_copy(h_hbm.at[srcv.at[0]], rows0, sem0)
        pltpu.async_copy(h_hbm.at[srcv.at[1]], rows1, sem1)
        guarded = (p + 1) * PCH > NCH

        def body(k, carry):
            j = 2 * k
            for b in range(2):
                g = p * PCH + j + b
                pltpu.make_async_copy(
                    h_hbm.at[srcv.at[0]], bufs[b], gsems[b]).wait()
                if guarded:
                    @pl.when(g < NCH)
                    def _():
                        pltpu.async_copy(
                            bufs[b], agg_sh.at[dstv.at[j + b]], ssems[b],
                            add=True)
                else:
                    pltpu.async_copy(
                        bufs[b], agg_sh.at[dstv.at[j + b]], ssems[b], add=True)
            for b in range(2):
                g = p * PCH + j + b
                if guarded:
                    @pl.when(g < NCH)
                    def _():
                        pltpu.make_async_copy(
                            bufs[b], agg_sh.at[dstv.at[0]], ssems[b]).wait()
                else:
                    pltpu.make_async_copy(
                        bufs[b], agg_sh.at[dstv.at[0]], ssems[b]).wait()

                @pl.when(j + 2 + b < PCH)
                def _():
                    pltpu.async_copy(
                        h_hbm.at[srcv.at[j + 2 + b]], bufs[b], gsems[b])
            return carry

        lax.fori_loop(0, PCH // 2, body, 0)
    plsc.subcore_barrier()
    pltpu.sync_copy(agg_sh.at[pl.ds(zrow, ZPT)],
                    out_hbm.at[pl.ds(c * N + zrow, ZPT)])

    @pl.when(s == NS - 1)
    def _():
        pltpu.sync_copy(agg_sh.at[pl.ds(tail, N - tail)],
                        out_hbm.at[pl.ds(c * N + tail, N - tail)])


@functools.lru_cache(maxsize=None)
def _segsum_call():
    return pl.kernel(
        _segsum_body,
        out_type=jax.ShapeDtypeStruct((2 * N, HALF), _f32),
        mesh=plsc.VectorSubcoreMesh(
            core_axis_name="c", subcore_axis_name="s", num_cores=NC,
            num_subcores=NS),
        scratch_types=[
            pltpu.VMEM((PCH, CHUNK), jnp.int32),
            pltpu.VMEM((PCH, CHUNK), jnp.int32),
            pltpu.VMEM((CHUNK, HALF), _f32),
            pltpu.VMEM((CHUNK, HALF), _f32),
            pltpu.VMEM_SHARED((NA, HALF), _f32),
            pltpu.SemaphoreType.DMA,
            pltpu.SemaphoreType.DMA,
            pltpu.SemaphoreType.DMA,
            pltpu.SemaphoreType.DMA,
        ],
    )


def _segment_sum(h_split, srcp, dstp, zeros):
    agg = _segsum_call()(h_split.reshape(2 * N, HALF), srcp, dstp, zeros)
    return agg.reshape(2, N, HALF)


def kernel(x, edge_index, seed_count, W_enc, b_enc, id_emb, W_self1, W_neigh1,
           b1, W_self2, W_neigh2, b2, ln_scale, ln_bias, W_head, b_head):
    seed = jnp.asarray(seed_count, jnp.int32).reshape(1)
    src = edge_index[0]
    dst = edge_index[1]
    src3 = src.reshape(NS, NCH, CHUNK)
    pad_rows = (jnp.arange(NS * (NCHP - NCH) * CHUNK, dtype=jnp.int32)
                % N).reshape(NS, NCHP - NCH, CHUNK)
    src3 = jnp.concatenate([src3, pad_rows], axis=1)
    srcp = jnp.stack([src3, src3 + N]).reshape(2 * NS * NPH, PCH, CHUNK)
    dst3 = dst.reshape(NS, NCH, CHUNK)
    dstp = jnp.concatenate(
        [dst3, jnp.full((NS, NCHP - NCH, CHUNK), N, jnp.int32)],
        axis=1).reshape(NS * NPH, PCH, CHUNK)
    zeros = jnp.zeros((NA, HALF), _f32)

    h0 = _encode(x, seed, W_enc, b_enc, id_emb)
    a1 = _segment_sum(h0, srcp, dstp, zeros)
    h1 = _sage(h0, a1, W_self1, W_neigh1, b1, relu=True)
    a2 = _segment_sum(h1, srcp, dstp, zeros)
    return _final(h1, a2, W_self2, W_neigh2, b2, ln_scale, ln_bias,
                  W_head, b_head)


# speedup vs baseline: 1.2341x; 1.2341x over previous
import functools

import jax
import jax.numpy as jnp
from jax import lax
from jax.experimental import pallas as pl
from jax.experimental.pallas import tpu as pltpu
from jax.experimental.pallas import tpu_sc as plsc

N = 10000
E = 160000
DIN = 128
C = 256
HALF = 128
OUT = 64

NC = 2
NS = 16
BN = 1000
NB = N // BN
EPT = E // NS
CHUNK = 128
NCHP = 80
NCHS = EPT // CHUNK + 1
NPH = 4
PCH = NCHP // NPH
NA = N + 8
ZPT = 624

_f32 = jnp.float32


def _enc_body(seed_ref, x_ref, we_ref, be_ref, ide_ref, out_ref):
    h = jnp.dot(x_ref[...], we_ref[...], preferred_element_type=_f32)
    h = h + be_ref[...]
    i = pl.program_id(0)
    rows = i * BN + lax.broadcasted_iota(jnp.int32, (BN, 1), 0)
    is_seed = (rows < seed_ref[0]).astype(_f32)
    h = h + is_seed * ide_ref[...]
    out_ref[0] = h[:, :HALF]
    out_ref[1] = h[:, HALF:]


def _encode(x, seed, W_enc, b_enc, id_emb):
    return pl.pallas_call(
        _enc_body,
        grid_spec=pltpu.PrefetchScalarGridSpec(
            num_scalar_prefetch=1,
            grid=(NB,),
            in_specs=[
                pl.BlockSpec((BN, DIN), lambda i, *_: (i, 0)),
                pl.BlockSpec((DIN, C), lambda i, *_: (0, 0)),
                pl.BlockSpec((1, C), lambda i, *_: (0, 0)),
                pl.BlockSpec((1, C), lambda i, *_: (0, 0)),
            ],
            out_specs=pl.BlockSpec((2, BN, HALF), lambda i, *_: (0, i, 0)),
        ),
        out_shape=jax.ShapeDtypeStruct((2, N, HALF), _f32),
    )(seed, x, W_enc, b_enc.reshape(1, C), id_emb)


def _sage_body(h_ref, a_ref, ws_ref, wn_ref, b_ref, out_ref, *, relu):
    z = jnp.dot(h_ref[0], ws_ref[:HALF, :], preferred_element_type=_f32)
    z += jnp.dot(h_ref[1], ws_ref[HALF:, :], preferred_element_type=_f32)
    z += jnp.dot(a_ref[0], wn_ref[:HALF, :], preferred_element_type=_f32)
    z += jnp.dot(a_ref[1], wn_ref[HALF:, :], preferred_element_type=_f32)
    z += b_ref[...]
    if relu:
        z = jnp.maximum(z, 0.0)
    out_ref[0] = z[:, :HALF]
    out_ref[1] = z[:, HALF:]


def _sage(h_split, a_split, W_self, W_neigh, b, relu):
    return pl.pallas_call(
        functools.partial(_sage_body, relu=relu),
        grid=(NB,),
        in_specs=[
            pl.BlockSpec((2, BN, HALF), lambda i: (0, i, 0)),
            pl.BlockSpec((2, BN, HALF), lambda i: (0, i, 0)),
            pl.BlockSpec((C, C), lambda i: (0, 0)),
            pl.BlockSpec((C, C), lambda i: (0, 0)),
            pl.BlockSpec((1, C), lambda i: (0, 0)),
        ],
        out_specs=pl.BlockSpec((2, BN, HALF), lambda i: (0, i, 0)),
        out_shape=jax.ShapeDtypeStruct((2, N, HALF), _f32),
    )(h_split, a_split, W_self, W_neigh, b.reshape(1, C))


def _final_body(h_ref, a_ref, ws_ref, wn_ref, b_ref, lns_ref, lnb_ref,
                wh_ref, bh_ref, out_ref):
    z = jnp.dot(h_ref[0], ws_ref[:HALF, :], preferred_element_type=_f32)
    z += jnp.dot(h_ref[1], ws_ref[HALF:, :], preferred_element_type=_f32)
    z += jnp.dot(a_ref[0], wn_ref[:HALF, :], preferred_element_type=_f32)
    z += jnp.dot(a_ref[1], wn_ref[HALF:, :], preferred_element_type=_f32)
    z += b_ref[...]
    mu = jnp.mean(z, axis=-1, keepdims=True)
    var = jnp.mean((z - mu) * (z - mu), axis=-1, keepdims=True)
    hn = (z - mu) * lax.rsqrt(var + 1e-5) * lns_ref[...] + lnb_ref[...]
    out_ref[...] = jnp.dot(hn, wh_ref[...], preferred_element_type=_f32)
    out_ref[...] += bh_ref[...]


def _final(h_split, a_split, W_self, W_neigh, b, ln_scale, ln_bias,
           W_head, b_head):
    return pl.pallas_call(
        _final_body,
        grid=(NB,),
        in_specs=[
            pl.BlockSpec((2, BN, HALF), lambda i: (0, i, 0)),
            pl.BlockSpec((2, BN, HALF), lambda i: (0, i, 0)),
            pl.BlockSpec((C, C), lambda i: (0, 0)),
            pl.BlockSpec((C, C), lambda i: (0, 0)),
            pl.BlockSpec((1, C), lambda i: (0, 0)),
            pl.BlockSpec((1, C), lambda i: (0, 0)),
            pl.BlockSpec((1, C), lambda i: (0, 0)),
            pl.BlockSpec((C, OUT), lambda i: (0, 0)),
            pl.BlockSpec((1, OUT), lambda i: (0, 0)),
        ],
        out_specs=pl.BlockSpec((BN, OUT), lambda i: (i, 0)),
        out_shape=jax.ShapeDtypeStruct((N, OUT), _f32),
    )(h_split, a_split, W_self, W_neigh, b.reshape(1, C),
      ln_scale.reshape(1, C), ln_bias.reshape(1, C), W_head,
      b_head.reshape(1, OUT))


def _segsum_body(h_hbm, srcp_hbm, dstp_hbm, zeros_hbm, out_hbm,
                 srcv, dstv, rows0, rows1, agg_sh, sem0, sem1):
    c = lax.axis_index("c")
    s = lax.axis_index("s")
    zrow = s * ZPT
    tail = NS * ZPT
    pltpu.sync_copy(zeros_hbm.at[pl.ds(zrow, ZPT)], agg_sh.at[pl.ds(zrow, ZPT)])

    @pl.when(s == NS - 1)
    def _():
        pltpu.sync_copy(zeros_hbm.at[pl.ds(tail, NA - tail)],
                        agg_sh.at[pl.ds(tail, NA - tail)])
    plsc.subcore_barrier()

    for p in range(NPH):
        pltpu.sync_copy(srcp_hbm.at[(c * NS + s) * NPH + p], srcv)
        pltpu.sync_copy(dstp_hbm.at[s * NPH + p], dstv)
        pltpu.async_copy(h_hbm.at[srcv.at[0]], rows0, sem0)
        guarded = (p + 1) * PCH > NCHS

        def body(k, carry):
            j = 2 * k
            g = p * PCH + j
            pltpu.make_async_copy(h_hbm.at[srcv.at[0]], rows0, sem0).wait()
            d1 = pltpu.async_copy(h_hbm.at[srcv.at[j + 1]], rows1, sem1)
            if guarded:
                @pl.when(g < NCHS)
                def _():
                    pltpu.sync_copy(rows0, agg_sh.at[dstv.at[j]], add=True)
            else:
                pltpu.sync_copy(rows0, agg_sh.at[dstv.at[j]], add=True)

            @pl.when(j + 2 < PCH)
            def _():
                pltpu.async_copy(h_hbm.at[srcv.at[j + 2]], rows0, sem0)

            d1.wait()
            if guarded:
                @pl.when(g + 1 < NCHS)
                def _():
                    pltpu.sync_copy(rows1, agg_sh.at[dstv.at[j + 1]], add=True)
            else:
                pltpu.sync_copy(rows1, agg_sh.at[dstv.at[j + 1]], add=True)
            return carry

        lax.fori_loop(0, PCH // 2, body, 0)
    plsc.subcore_barrier()
    pltpu.sync_copy(agg_sh.at[pl.ds(zrow, ZPT)],
                    out_hbm.at[pl.ds(c * N + zrow, ZPT)])

    @pl.when(s == NS - 1)
    def _():
        pltpu.sync_copy(agg_sh.at[pl.ds(tail, N - tail)],
                        out_hbm.at[pl.ds(c * N + tail, N - tail)])


@functools.lru_cache(maxsize=None)
def _segsum_call():
    return pl.kernel(
        _segsum_body,
        out_type=jax.ShapeDtypeStruct((2 * N, HALF), _f32),
        mesh=plsc.VectorSubcoreMesh(
            core_axis_name="c", subcore_axis_name="s", num_cores=NC,
            num_subcores=NS),
        scratch_types=[
            pltpu.VMEM((PCH, CHUNK), jnp.int32),
            pltpu.VMEM((PCH, CHUNK), jnp.int32),
            pltpu.VMEM((CHUNK, HALF), _f32),
            pltpu.VMEM((CHUNK, HALF), _f32),
            pltpu.VMEM_SHARED((NA, HALF), _f32),
            pltpu.SemaphoreType.DMA,
            pltpu.SemaphoreType.DMA,
        ],
    )


def _segment_sum(h_split, srcp, dstp, zeros):
    agg = _segsum_call()(h_split.reshape(2 * N, HALF), srcp, dstp, zeros)
    return agg.reshape(2, N, HALF)


def kernel(x, edge_index, seed_count, W_enc, b_enc, id_emb, W_self1, W_neigh1,
           b1, W_self2, W_neigh2, b2, ln_scale, ln_bias, W_head, b_head):
    seed = jnp.asarray(seed_count, jnp.int32).reshape(1)
    src = edge_index[0]
    dst = edge_index[1]
    npad = NCHP * CHUNK - EPT
    pad_rows = (jnp.arange(NS * npad, dtype=jnp.int32) % N).reshape(NS, npad)
    src2 = jnp.concatenate([src.reshape(NS, EPT), pad_rows], axis=1)
    srcp = jnp.stack([src2, src2 + N]).reshape(2 * NS * NPH, PCH, CHUNK)
    pad_dump = N + (jnp.arange(NS * npad, dtype=jnp.int32) % 8).reshape(
        NS, npad)
    dstp = jnp.concatenate([dst.reshape(NS, EPT), pad_dump],
                           axis=1).reshape(NS * NPH, PCH, CHUNK)
    zeros = jnp.zeros((NA, HALF), _f32)

    h0 = _encode(x, seed, W_enc, b_enc, id_emb)
    a1 = _segment_sum(h0, srcp, dstp, zeros)
    h1 = _sage(h0, a1, W_self1, W_neigh1, b1, relu=True)
    a2 = _segment_sum(h1, srcp, dstp, zeros)
    return _final(h1, a2, W_self2, W_neigh2, b2, ln_scale, ln_bias,
                  W_head, b_head)


# speedup vs baseline: 1.3079x; 1.0598x over previous
import functools

import jax
import jax.numpy as jnp
from jax import lax
from jax.experimental import pallas as pl
from jax.experimental.pallas import tpu as pltpu
from jax.experimental.pallas import tpu_sc as plsc

N = 10000
E = 160000
DIN = 128
C = 256
HALF = 128
OUT = 64

NC = 2
NS = 16
BN = 2000
NB = N // BN
EPT = E // NS
CHUNK = 128
NCHP = 80
NCHS = EPT // CHUNK + 1
NPH = 4
PCH = NCHP // NPH
NA = N + 8
ZPT = 624

_f32 = jnp.float32


def _enc_body(seed_ref, x_ref, we_ref, be_ref, ide_ref, out_ref):
    h = jnp.dot(x_ref[...], we_ref[...], preferred_element_type=_f32)
    h = h + be_ref[...]
    i = pl.program_id(0)
    rows = i * BN + lax.broadcasted_iota(jnp.int32, (BN, 1), 0)
    is_seed = (rows < seed_ref[0]).astype(_f32)
    h = h + is_seed * ide_ref[...]
    out_ref[0] = h[:, :HALF]
    out_ref[1] = h[:, HALF:]


def _encode(x, seed, W_enc, b_enc, id_emb):
    return pl.pallas_call(
        _enc_body,
        grid_spec=pltpu.PrefetchScalarGridSpec(
            num_scalar_prefetch=1,
            grid=(NB,),
            in_specs=[
                pl.BlockSpec((BN, DIN), lambda i, *_: (i, 0)),
                pl.BlockSpec((DIN, C), lambda i, *_: (0, 0)),
                pl.BlockSpec((1, C), lambda i, *_: (0, 0)),
                pl.BlockSpec((1, C), lambda i, *_: (0, 0)),
            ],
            out_specs=pl.BlockSpec((2, BN, HALF), lambda i, *_: (0, i, 0)),
        ),
        out_shape=jax.ShapeDtypeStruct((2, N, HALF), _f32),
    )(seed, x, W_enc, b_enc.reshape(1, C), id_emb)


def _sage_body(h_ref, a_ref, ws_ref, wn_ref, b_ref, out_ref, *, relu):
    z = jnp.dot(h_ref[0], ws_ref[:HALF, :], preferred_element_type=_f32)
    z += jnp.dot(h_ref[1], ws_ref[HALF:, :], preferred_element_type=_f32)
    z += jnp.dot(a_ref[0], wn_ref[:HALF, :], preferred_element_type=_f32)
    z += jnp.dot(a_ref[1], wn_ref[HALF:, :], preferred_element_type=_f32)
    z += b_ref[...]
    if relu:
        z = jnp.maximum(z, 0.0)
    out_ref[0] = z[:, :HALF]
    out_ref[1] = z[:, HALF:]


def _sage(h_split, a_split, W_self, W_neigh, b, relu):
    return pl.pallas_call(
        functools.partial(_sage_body, relu=relu),
        grid=(NB,),
        in_specs=[
            pl.BlockSpec((2, BN, HALF), lambda i: (0, i, 0)),
            pl.BlockSpec((2, BN, HALF), lambda i: (0, i, 0)),
            pl.BlockSpec((C, C), lambda i: (0, 0)),
            pl.BlockSpec((C, C), lambda i: (0, 0)),
            pl.BlockSpec((1, C), lambda i: (0, 0)),
        ],
        out_specs=pl.BlockSpec((2, BN, HALF), lambda i: (0, i, 0)),
        out_shape=jax.ShapeDtypeStruct((2, N, HALF), _f32),
    )(h_split, a_split, W_self, W_neigh, b.reshape(1, C))


def _final_body(h_ref, a_ref, ws_ref, wn_ref, b_ref, lns_ref, lnb_ref,
                wh_ref, bh_ref, out_ref):
    z = jnp.dot(h_ref[0], ws_ref[:HALF, :], preferred_element_type=_f32)
    z += jnp.dot(h_ref[1], ws_ref[HALF:, :], preferred_element_type=_f32)
    z += jnp.dot(a_ref[0], wn_ref[:HALF, :], preferred_element_type=_f32)
    z += jnp.dot(a_ref[1], wn_ref[HALF:, :], preferred_element_type=_f32)
    z += b_ref[...]
    mu = jnp.mean(z, axis=-1, keepdims=True)
    var = jnp.mean((z - mu) * (z - mu), axis=-1, keepdims=True)
    hn = (z - mu) * lax.rsqrt(var + 1e-5) * lns_ref[...] + lnb_ref[...]
    out_ref[...] = jnp.dot(hn, wh_ref[...], preferred_element_type=_f32)
    out_ref[...] += bh_ref[...]


def _final(h_split, a_split, W_self, W_neigh, b, ln_scale, ln_bias,
           W_head, b_head):
    return pl.pallas_call(
        _final_body,
        grid=(NB,),
        in_specs=[
            pl.BlockSpec((2, BN, HALF), lambda i: (0, i, 0)),
            pl.BlockSpec((2, BN, HALF), lambda i: (0, i, 0)),
            pl.BlockSpec((C, C), lambda i: (0, 0)),
            pl.BlockSpec((C, C), lambda i: (0, 0)),
            pl.BlockSpec((1, C), lambda i: (0, 0)),
            pl.BlockSpec((1, C), lambda i: (0, 0)),
            pl.BlockSpec((1, C), lambda i: (0, 0)),
            pl.BlockSpec((C, OUT), lambda i: (0, 0)),
            pl.BlockSpec((1, OUT), lambda i: (0, 0)),
        ],
        out_specs=pl.BlockSpec((BN, OUT), lambda i: (i, 0)),
        out_shape=jax.ShapeDtypeStruct((N, OUT), _f32),
    )(h_split, a_split, W_self, W_neigh, b.reshape(1, C),
      ln_scale.reshape(1, C), ln_bias.reshape(1, C), W_head,
      b_head.reshape(1, OUT))


def _segsum_body(h_hbm, srcp_hbm, dstp_hbm, out_hbm,
                 srcv, dstv, rows0, rows1, agg_sh, sem0, sem1):
    c = lax.axis_index("c")
    s = lax.axis_index("s")
    zrow = s * ZPT
    tail = NS * ZPT

    zv = jnp.zeros((16,), _f32)

    def zbody(i, carry):
        for kk in range(HALF // 16):
            rows0[i, pl.ds(kk * 16, 16)] = zv
        return carry

    lax.fori_loop(0, CHUNK, zbody, 0)
    for m in range(ZPT // CHUNK):
        pltpu.sync_copy(rows0, agg_sh.at[pl.ds(zrow + m * CHUNK, CHUNK)])
    rem = ZPT % CHUNK
    pltpu.sync_copy(rows0.at[pl.ds(0, rem)],
                    agg_sh.at[pl.ds(zrow + ZPT - rem, rem)])

    @pl.when(s == NS - 1)
    def _():
        pltpu.sync_copy(rows0.at[pl.ds(0, NA - tail)],
                        agg_sh.at[pl.ds(tail, NA - tail)])
    plsc.subcore_barrier()

    hc = h_hbm.at[c]
    for p in range(NPH):
        pltpu.sync_copy(srcp_hbm.at[s * NPH + p], srcv)
        pltpu.sync_copy(dstp_hbm.at[s * NPH + p], dstv)
        pltpu.async_copy(hc.at[srcv.at[0]], rows0, sem0)
        guarded = (p + 1) * PCH > NCHS

        def body(k, carry):
            j = 2 * k
            g = p * PCH + j
            pltpu.make_async_copy(hc.at[srcv.at[0]], rows0, sem0).wait()
            d1 = pltpu.async_copy(hc.at[srcv.at[j + 1]], rows1, sem1)
            if guarded:
                @pl.when(g < NCHS)
                def _():
                    pltpu.sync_copy(rows0, agg_sh.at[dstv.at[j]], add=True)
            else:
                pltpu.sync_copy(rows0, agg_sh.at[dstv.at[j]], add=True)

            @pl.when(j + 2 < PCH)
            def _():
                pltpu.async_copy(hc.at[srcv.at[j + 2]], rows0, sem0)

            d1.wait()
            if guarded:
                @pl.when(g + 1 < NCHS)
                def _():
                    pltpu.sync_copy(rows1, agg_sh.at[dstv.at[j + 1]], add=True)
            else:
                pltpu.sync_copy(rows1, agg_sh.at[dstv.at[j + 1]], add=True)
            return carry

        lax.fori_loop(0, PCH // 2, body, 0)
    plsc.subcore_barrier()
    pltpu.sync_copy(agg_sh.at[pl.ds(zrow, ZPT)],
                    out_hbm.at[pl.ds(c * N + zrow, ZPT)])

    @pl.when(s == NS - 1)
    def _():
        pltpu.sync_copy(agg_sh.at[pl.ds(tail, N - tail)],
                        out_hbm.at[pl.ds(c * N + tail, N - tail)])


@functools.lru_cache(maxsize=None)
def _segsum_call():
    return pl.kernel(
        _segsum_body,
        out_type=jax.ShapeDtypeStruct((2 * N, HALF), _f32),
        name="segsum",
        mesh=plsc.VectorSubcoreMesh(
            core_axis_name="c", subcore_axis_name="s", num_cores=NC,
            num_subcores=NS),
        scratch_types=[
            pltpu.VMEM((PCH, CHUNK), jnp.int32),
            pltpu.VMEM((PCH, CHUNK), jnp.int32),
            pltpu.VMEM((CHUNK, HALF), _f32),
            pltpu.VMEM((CHUNK, HALF), _f32),
            pltpu.VMEM_SHARED((NA, HALF), _f32),
            pltpu.SemaphoreType.DMA,
            pltpu.SemaphoreType.DMA,
        ],
    )


def _segment_sum(h_split, srcp, dstp):
    agg = _segsum_call()(h_split, srcp, dstp)
    return agg.reshape(2, N, HALF)


def kernel(x, edge_index, seed_count, W_enc, b_enc, id_emb, W_self1, W_neigh1,
           b1, W_self2, W_neigh2, b2, ln_scale, ln_bias, W_head, b_head):
    seed = jnp.asarray(seed_count, jnp.int32).reshape(1)
    src = edge_index[0]
    dst = edge_index[1]
    npad = NCHP * CHUNK - EPT
    pad_rows = (jnp.arange(NS * npad, dtype=jnp.int32) % N).reshape(NS, npad)
    srcp = jnp.concatenate([src.reshape(NS, EPT), pad_rows],
                           axis=1).reshape(NS * NPH, PCH, CHUNK)
    pad_dump = N + (jnp.arange(NS * npad, dtype=jnp.int32) % 8).reshape(
        NS, npad)
    dstp = jnp.concatenate([dst.reshape(NS, EPT), pad_dump],
                           axis=1).reshape(NS * NPH, PCH, CHUNK)

    h0 = _encode(x, seed, W_enc, b_enc, id_emb)
    a1 = _segment_sum(h0, srcp, dstp)
    h1 = _sage(h0, a1, W_self1, W_neigh1, b1, relu=True)
    a2 = _segment_sum(h1, srcp, dstp)
    return _final(h1, a2, W_self2, W_neigh2, b2, ln_scale, ln_bias,
                  W_head, b_head)


# speedup vs baseline: 1.3388x; 1.0237x over previous
import functools

import jax
import jax.numpy as jnp
from jax import lax
from jax.experimental import pallas as pl
from jax.experimental.pallas import tpu as pltpu
from jax.experimental.pallas import tpu_sc as plsc

N = 10000
E = 160000
DIN = 128
C = 256
HALF = 128
OUT = 64

NC = 2
NS = 16
BN = 2000
NB = N // BN
EPT = E // NS
CHUNK = 128
NCHP = 80
NCHS = EPT // CHUNK + 1
NPH = 4
PCH = NCHP // NPH
NA = N + 8
ZPT = 624

_f32 = jnp.float32


def _enc_body(seed_ref, x_ref, we_ref, be_ref, ide_ref, out_ref):
    h = jnp.dot(x_ref[...], we_ref[...], preferred_element_type=_f32)
    h = h + be_ref[...]
    i = pl.program_id(0)
    rows = i * BN + lax.broadcasted_iota(jnp.int32, (BN, 1), 0)
    is_seed = (rows < seed_ref[0]).astype(_f32)
    h = h + is_seed * ide_ref[...]
    out_ref[0] = h[:, :HALF]
    out_ref[1] = h[:, HALF:]


def _encode(x, seed, W_enc, b_enc, id_emb):
    return pl.pallas_call(
        _enc_body,
        grid_spec=pltpu.PrefetchScalarGridSpec(
            num_scalar_prefetch=1,
            grid=(NB,),
            in_specs=[
                pl.BlockSpec((BN, DIN), lambda i, *_: (i, 0)),
                pl.BlockSpec((DIN, C), lambda i, *_: (0, 0)),
                pl.BlockSpec((1, C), lambda i, *_: (0, 0)),
                pl.BlockSpec((1, C), lambda i, *_: (0, 0)),
            ],
            out_specs=pl.BlockSpec((2, BN, HALF), lambda i, *_: (0, i, 0)),
        ),
        out_shape=jax.ShapeDtypeStruct((2, N, HALF), _f32),
    )(seed, x, W_enc, b_enc.reshape(1, C), id_emb)


def _sage_body(h_ref, a_ref, ws_ref, wn_ref, b_ref, out_ref, *, relu):
    z = jnp.dot(h_ref[0], ws_ref[:HALF, :], preferred_element_type=_f32)
    z += jnp.dot(h_ref[1], ws_ref[HALF:, :], preferred_element_type=_f32)
    z += jnp.dot(a_ref[0], wn_ref[:HALF, :], preferred_element_type=_f32)
    z += jnp.dot(a_ref[1], wn_ref[HALF:, :], preferred_element_type=_f32)
    z += b_ref[...]
    if relu:
        z = jnp.maximum(z, 0.0)
    out_ref[0] = z[:, :HALF]
    out_ref[1] = z[:, HALF:]


def _sage(h_split, a_split, W_self, W_neigh, b, relu):
    return pl.pallas_call(
        functools.partial(_sage_body, relu=relu),
        grid=(NB,),
        in_specs=[
            pl.BlockSpec((2, BN, HALF), lambda i: (0, i, 0)),
            pl.BlockSpec((2, BN, HALF), lambda i: (0, i, 0)),
            pl.BlockSpec((C, C), lambda i: (0, 0)),
            pl.BlockSpec((C, C), lambda i: (0, 0)),
            pl.BlockSpec((1, C), lambda i: (0, 0)),
        ],
        out_specs=pl.BlockSpec((2, BN, HALF), lambda i: (0, i, 0)),
        out_shape=jax.ShapeDtypeStruct((2, N, HALF), _f32),
    )(h_split, a_split, W_self, W_neigh, b.reshape(1, C))


def _final_body(h_ref, a_ref, ws_ref, wn_ref, b_ref, lns_ref, lnb_ref,
                wh_ref, bh_ref, out_ref):
    z = jnp.dot(h_ref[0], ws_ref[:HALF, :], preferred_element_type=_f32)
    z += jnp.dot(h_ref[1], ws_ref[HALF:, :], preferred_element_type=_f32)
    z += jnp.dot(a_ref[0], wn_ref[:HALF, :], preferred_element_type=_f32)
    z += jnp.dot(a_ref[1], wn_ref[HALF:, :], preferred_element_type=_f32)
    z += b_ref[...]
    mu = jnp.mean(z, axis=-1, keepdims=True)
    var = jnp.mean((z - mu) * (z - mu), axis=-1, keepdims=True)
    hn = (z - mu) * lax.rsqrt(var + 1e-5) * lns_ref[...] + lnb_ref[...]
    out_ref[...] = jnp.dot(hn, wh_ref[...], preferred_element_type=_f32)
    out_ref[...] += bh_ref[...]


def _final(h_split, a_split, W_self, W_neigh, b, ln_scale, ln_bias,
           W_head, b_head):
    return pl.pallas_call(
        _final_body,
        grid=(NB,),
        in_specs=[
            pl.BlockSpec((2, BN, HALF), lambda i: (0, i, 0)),
            pl.BlockSpec((2, BN, HALF), lambda i: (0, i, 0)),
            pl.BlockSpec((C, C), lambda i: (0, 0)),
            pl.BlockSpec((C, C), lambda i: (0, 0)),
            pl.BlockSpec((1, C), lambda i: (0, 0)),
            pl.BlockSpec((1, C), lambda i: (0, 0)),
            pl.BlockSpec((1, C), lambda i: (0, 0)),
            pl.BlockSpec((C, OUT), lambda i: (0, 0)),
            pl.BlockSpec((1, OUT), lambda i: (0, 0)),
        ],
        out_specs=pl.BlockSpec((BN, OUT), lambda i: (i, 0)),
        out_shape=jax.ShapeDtypeStruct((N, OUT), _f32),
    )(h_split, a_split, W_self, W_neigh, b.reshape(1, C),
      ln_scale.reshape(1, C), ln_bias.reshape(1, C), W_head,
      b_head.reshape(1, OUT))


def _segsum_body(h_hbm, edg_hbm, out_hbm,
                 srcv, dstv, rows0, rows1, agg_sh, sem0, sem1):
    c = lax.axis_index("c")
    s = lax.axis_index("s")
    zrow = s * ZPT
    tail = NS * ZPT

    zv = jnp.zeros((16,), _f32)

    def zbody(i, carry):
        for kk in range(HALF // 16):
            rows0[i, pl.ds(kk * 16, 16)] = zv
        return carry

    lax.fori_loop(0, CHUNK, zbody, 0)
    for m in range(ZPT // CHUNK):
        pltpu.sync_copy(rows0, agg_sh.at[pl.ds(zrow + m * CHUNK, CHUNK)])
    rem = ZPT % CHUNK
    pltpu.sync_copy(rows0.at[pl.ds(0, rem)],
                    agg_sh.at[pl.ds(zrow + ZPT - rem, rem)])

    @pl.when(s == NS - 1)
    def _():
        pltpu.sync_copy(rows0.at[pl.ds(0, NA - tail)],
                        agg_sh.at[pl.ds(tail, NA - tail)])
    plsc.subcore_barrier()

    hc = h_hbm.at[c]
    for p in range(NPH):
        pltpu.sync_copy(edg_hbm.at[s * NPH + p], srcv)
        pltpu.sync_copy(edg_hbm.at[(NS + s) * NPH + p], dstv)
        pltpu.async_copy(hc.at[srcv.at[0]], rows0, sem0)
        guarded = (p + 1) * PCH > NCHS

        def body(k, carry):
            j = 2 * k
            g = p * PCH + j
            pltpu.make_async_copy(hc.at[srcv.at[0]], rows0, sem0).wait()
            d1 = pltpu.async_copy(hc.at[srcv.at[j + 1]], rows1, sem1)
            if guarded:
                @pl.when(g < NCHS)
                def _():
                    pltpu.sync_copy(rows0, agg_sh.at[dstv.at[j]], add=True)
            else:
                pltpu.sync_copy(rows0, agg_sh.at[dstv.at[j]], add=True)

            @pl.when(j + 2 < PCH)
            def _():
                pltpu.async_copy(hc.at[srcv.at[j + 2]], rows0, sem0)

            d1.wait()
            if guarded:
                @pl.when(g + 1 < NCHS)
                def _():
                    pltpu.sync_copy(rows1, agg_sh.at[dstv.at[j + 1]], add=True)
            else:
                pltpu.sync_copy(rows1, agg_sh.at[dstv.at[j + 1]], add=True)
            return carry

        lax.fori_loop(0, PCH // 2, body, 0)
    plsc.subcore_barrier()
    pltpu.sync_copy(agg_sh.at[pl.ds(zrow, ZPT)],
                    out_hbm.at[pl.ds(c * N + zrow, ZPT)])

    @pl.when(s == NS - 1)
    def _():
        pltpu.sync_copy(agg_sh.at[pl.ds(tail, N - tail)],
                        out_hbm.at[pl.ds(c * N + tail, N - tail)])


@functools.lru_cache(maxsize=None)
def _segsum_call():
    return pl.kernel(
        _segsum_body,
        out_type=jax.ShapeDtypeStruct((2 * N, HALF), _f32),
        name="segsum",
        mesh=plsc.VectorSubcoreMesh(
            core_axis_name="c", subcore_axis_name="s", num_cores=NC,
            num_subcores=NS),
        scratch_types=[
            pltpu.VMEM((PCH, CHUNK), jnp.int32),
            pltpu.VMEM((PCH, CHUNK), jnp.int32),
            pltpu.VMEM((CHUNK, HALF), _f32),
            pltpu.VMEM((CHUNK, HALF), _f32),
            pltpu.VMEM_SHARED((NA, HALF), _f32),
            pltpu.SemaphoreType.DMA,
            pltpu.SemaphoreType.DMA,
        ],
    )


def _segment_sum(h_split, eip):
    agg = _segsum_call()(h_split, eip)
    return agg.reshape(2, N, HALF)


def kernel(x, edge_index, seed_count, W_enc, b_enc, id_emb, W_self1, W_neigh1,
           b1, W_self2, W_neigh2, b2, ln_scale, ln_bias, W_head, b_head):
    seed = jnp.asarray(seed_count, jnp.int32).reshape(1)
    npad = NCHP * CHUNK - EPT
    pad_rows = (jnp.arange(NS * npad, dtype=jnp.int32) % N).reshape(1, NS, npad)
    pad_dump = N + (jnp.arange(NS * npad, dtype=jnp.int32) % 8).reshape(
        1, NS, npad)
    pads = jnp.concatenate([pad_rows, pad_dump], axis=0)
    eip = jnp.concatenate([edge_index.reshape(2, NS, EPT), pads],
                          axis=2).reshape(2 * NS * NPH, PCH, CHUNK)

    h0 = _encode(x, seed, W_enc, b_enc, id_emb)
    a1 = _segment_sum(h0, eip)
    h1 = _sage(h0, a1, W_self1, W_neigh1, b1, relu=True)
    a2 = _segment_sum(h1, eip)
    return _final(h1, a2, W_self2, W_neigh2, b2, ln_scale, ln_bias,
                  W_head, b_head)
